# baseline (speedup 1.0000x reference)
import jax
import jax.numpy as jnp
from jax import lax
from jax.experimental import pallas as pl
from jax.experimental.pallas import tpu as pltpu
from jax.experimental.pallas import tpu_sc as plsc

N = 10000
E = 320000
D = 128

NC = 2
NS = 16
NW = NC * NS
EPW = E // NW
CHUNK = 128
NGRP = 10
EPW_P = NGRP * 8 * CHUNK
PAD = EPW_P - EPW
ACC_ROWS = 10048
TRASH = 10040
HROWS = 80
NFL = 125

_MESH = plsc.VectorSubcoreMesh(
    core_axis_name="c", subcore_axis_name="s", num_cores=NC, num_subcores=NS
)
_SC_PARAMS = pltpu.CompilerParams(needs_layout_passes=False)


def _degree_body(src_hbm, dst_hbm, out_s, out_d, sidx, didx, ps, pd):
    cid = lax.axis_index("c")
    sid = lax.axis_index("s")
    wid = cid * NS + sid
    pltpu.sync_copy(src_hbm.at[wid], sidx)
    pltpu.sync_copy(dst_hbm.at[wid], didx)

    zeros = jnp.zeros((16,), jnp.float32)

    @pl.loop(0, HROWS)
    def _zero(i):
        for j in range(8):
            ps[i, pl.ds(j * 16, 16)] = zeros
            pd[i, pl.ds(j * 16, 16)] = zeros

    ones = jnp.ones((16,), jnp.float32)
    m127 = jnp.full((16,), 127, jnp.int32)

    @pl.loop(0, HROWS)
    def _hist(i):
        for j in range(8):
            s = sidx[i, pl.ds(j * 16, 16)]
            d = didx[i, pl.ds(j * 16, 16)]
            plsc.addupdate_scatter(ps, [s >> 7, s & m127], ones)
            plsc.addupdate_scatter(pd, [d >> 7, d & m127], ones)

    pltpu.sync_copy(ps, out_s.at[wid])
    pltpu.sync_copy(pd, out_d.at[wid])


_degree = pl.kernel(
    _degree_body,
    out_type=(
        jax.ShapeDtypeStruct((NW, HROWS, 128), jnp.float32),
        jax.ShapeDtypeStruct((NW, HROWS, 128), jnp.float32),
    ),
    mesh=_MESH,
    compiler_params=_SC_PARAMS,
    scratch_types=[
        pltpu.VMEM((HROWS, 128), jnp.int32),
        pltpu.VMEM((HROWS, 128), jnp.int32),
        pltpu.VMEM((HROWS, 128), jnp.float32),
        pltpu.VMEM((HROWS, 128), jnp.float32),
    ],
)


def _agg_body(table, src_hbm, dst_hbm, out, shared, si, di0, di1,
              rows0, rows1, semg0, semg1, semsc0, semsc1, semi0, semi1):
    cid = lax.axis_index("c")
    sid = lax.axis_index("s")
    wid = cid * NS + sid

    @pl.loop(0, NGRP)
    def _stg(g):
        pltpu.sync_copy(src_hbm.at[wid, g], si.at[pl.ds(g * 8, 8)])

    zeros = jnp.zeros((16,), jnp.float32)

    @pl.loop(0, CHUNK)
    def _z(i):
        for j in range(8):
            rows0[i, pl.ds(j * 16, 16)] = zeros

    @pl.loop(0, 8)
    def _zs(m):
        ch = sid + m * NS

        @pl.when(ch * 80 < ACC_ROWS)
        def _():
            pltpu.sync_copy(rows0.at[pl.ds(0, 80)], shared.at[pl.ds(ch * 80, 80)])

    plsc.subcore_barrier()

    bufs = (rows0, rows1)
    semg = (semg0, semg1)
    semsc = (semsc0, semsc1)

    def startG(b, c):
        pltpu.async_copy(table.at[si.at[c]], bufs[b], semg[b])

    def waitG(b, c):
        pltpu.make_async_copy(table.at[si.at[c]], bufs[b], semg[b]).wait()

    def startS(b, didx):
        pltpu.async_copy(bufs[b], shared.at[didx], semsc[b], add=True)

    def waitS(b, didx):
        pltpu.make_async_copy(bufs[b], shared.at[didx], semsc[b]).wait()

    pltpu.async_copy(dst_hbm.at[wid, 0], di0, semi0)
    startG(0, 0)

    @pl.loop(0, 5)
    def _pair(it):
        c0 = it * 16
        for r2 in range(16):
            c = c0 + r2
            b = r2 % 2
            ob = 1 - b
            dcur = di0 if r2 < 8 else di1
            ridx = r2 % 8
            if r2 == 0:
                pltpu.make_async_copy(dst_hbm.at[wid, 0], di0, semi0).wait()
            if r2 == 8:
                pltpu.make_async_copy(dst_hbm.at[wid, 0], di1, semi1).wait()
            waitG(b, c)
            startS(b, dcur.at[ridx])
            if r2 == 0:
                @pl.when(c > 0)
                def _():
                    waitS(1, di1.at[7])
            else:
                pdcur = di0 if (r2 - 1) < 8 else di1
                waitS(ob, pdcur.at[(r2 - 1) % 8])

            @pl.when(c + 1 < NGRP * 8)
            def _():
                startG(ob, c + 1)

            if r2 == 2:
                pltpu.async_copy(dst_hbm.at[wid, 2 * it + 1], di1, semi1)
            if r2 == 10:
                @pl.when(2 * it + 2 < NGRP)
                def _():
                    pltpu.async_copy(dst_hbm.at[wid, 2 * it + 2], di0, semi0)

    waitS(1, di1.at[7])

    plsc.subcore_barrier()

    @pl.loop(0, 8)
    def _fl(m):
        ch = sid + m * NS

        @pl.when(ch < NFL)
        def _():
            pltpu.sync_copy(shared.at[pl.ds(ch * 80, 80)], rows0.at[pl.ds(0, 80)])
            pltpu.sync_copy(rows0.at[pl.ds(0, 80)], out.at[cid, pl.ds(ch * 80, 80)])


_aggregate = pl.kernel(
    _agg_body,
    out_type=jax.ShapeDtypeStruct((NC, N, D), jnp.float32),
    mesh=_MESH,
    compiler_params=_SC_PARAMS,
    scratch_types=[
        pltpu.VMEM_SHARED((ACC_ROWS, D), jnp.float32),
        pltpu.VMEM((NGRP * 8, CHUNK), jnp.int32),
        pltpu.VMEM((8, CHUNK), jnp.int32),
        pltpu.VMEM((8, CHUNK), jnp.int32),
        pltpu.VMEM((CHUNK, D), jnp.float32),
        pltpu.VMEM((CHUNK, D), jnp.float32),
        pltpu.SemaphoreType.DMA,
        pltpu.SemaphoreType.DMA,
        pltpu.SemaphoreType.DMA,
        pltpu.SemaphoreType.DMA,
        pltpu.SemaphoreType.DMA,
        pltpu.SemaphoreType.DMA,
    ],
)


_RB = 1000
_NB = N // _RB
_SRC_PAD_COUNT = float(NW * PAD)


def _scale_body(x_ref, ds_ref, dd_ref, xs_ref, ns_ref, nd_ref):
    i = pl.program_id(0)
    ds = jnp.sum(ds_ref[...], axis=1, keepdims=True)
    dd = jnp.sum(dd_ref[...], axis=1, keepdims=True)
    row0 = (lax.broadcasted_iota(jnp.int32, (_RB, 1), 0) == 0) & (i == 0)
    ds = ds - jnp.where(row0, _SRC_PAD_COUNT, 0.0)
    ns = lax.rsqrt(jnp.maximum(ds, 1.0))
    nd = lax.rsqrt(jnp.maximum(dd, 1.0))
    xs_ref[...] = x_ref[...] * ns
    ns_ref[...] = ns
    nd_ref[...] = nd


def _scale(x, ds_t, dd_t):
    return pl.pallas_call(
        _scale_body,
        grid=(_NB,),
        in_specs=[
            pl.BlockSpec((_RB, D), lambda i: (i, 0)),
            pl.BlockSpec((_RB, NW), lambda i: (i, 0)),
            pl.BlockSpec((_RB, NW), lambda i: (i, 0)),
        ],
        out_specs=[
            pl.BlockSpec((_RB, D), lambda i: (i, 0)),
            pl.BlockSpec((_RB, 1), lambda i: (i, 0)),
            pl.BlockSpec((_RB, 1), lambda i: (i, 0)),
        ],
        out_shape=[
            jax.ShapeDtypeStruct((N, D), jnp.float32),
            jax.ShapeDtypeStruct((N, 1), jnp.float32),
            jax.ShapeDtypeStruct((N, 1), jnp.float32),
        ],
    )(x, ds_t, dd_t)


def _dense1_body(p1a, p1b, ns, nd, w1, b1, w2, h1_o, t2_o):
    agg = (p1a[...] + p1b[...]) * nd[...]
    h1 = jnp.maximum(
        jnp.dot(agg, w1[...], preferred_element_type=jnp.float32) + b1[...], 0.0
    )
    h1_o[...] = h1
    t2_o[...] = ns[...] * jnp.dot(h1, w2[...], preferred_element_type=jnp.float32)


def _dense1(p1a, p1b, ns, nd, w1, b1, w2):
    return pl.pallas_call(
        _dense1_body,
        grid=(_NB,),
        in_specs=[
            pl.BlockSpec((_RB, D), lambda i: (i, 0)),
            pl.BlockSpec((_RB, D), lambda i: (i, 0)),
            pl.BlockSpec((_RB, 1), lambda i: (i, 0)),
            pl.BlockSpec((_RB, 1), lambda i: (i, 0)),
            pl.BlockSpec((D, D), lambda i: (0, 0)),
            pl.BlockSpec((1, D), lambda i: (0, 0)),
            pl.BlockSpec((D, D), lambda i: (0, 0)),
        ],
        out_specs=[
            pl.BlockSpec((_RB, D), lambda i: (i, 0)),
            pl.BlockSpec((_RB, D), lambda i: (i, 0)),
        ],
        out_shape=[
            jax.ShapeDtypeStruct((N, D), jnp.float32),
            jax.ShapeDtypeStruct((N, D), jnp.float32),
        ],
    )(p1a, p1b, ns, nd, w1, b1, w2)


def _dense2_body(p2a, p2b, nd, b2, h2_o):
    h2_o[...] = (p2a[...] + p2b[...]) * nd[...] + b2[...]


def _dense2(p2a, p2b, nd, b2):
    return pl.pallas_call(
        _dense2_body,
        grid=(_NB,),
        in_specs=[
            pl.BlockSpec((_RB, D), lambda i: (i, 0)),
            pl.BlockSpec((_RB, D), lambda i: (i, 0)),
            pl.BlockSpec((_RB, 1), lambda i: (i, 0)),
            pl.BlockSpec((1, D), lambda i: (0, 0)),
        ],
        out_specs=pl.BlockSpec((_RB, D), lambda i: (i, 0)),
        out_shape=jax.ShapeDtypeStruct((N, D), jnp.float32),
    )(p2a, p2b, nd, b2)


@jax.jit
def kernel(feats, edge_index, W1, b1, W2, b2):
    src = edge_index[0].reshape(NW, EPW)
    dst = edge_index[1].reshape(NW, EPW)
    src_p = jnp.pad(src, ((0, 0), (0, PAD)), constant_values=0)
    dst_p = jnp.pad(dst, ((0, 0), (0, PAD)), constant_values=TRASH)
    src4 = src_p.reshape(NW, NGRP, 8, CHUNK)
    dst4 = dst_p.reshape(NW, NGRP, 8, CHUNK)
    src_h = src_p.reshape(NW, HROWS, 128)
    dst_h = dst_p.reshape(NW, HROWS, 128)

    dS_p, dD_p = _degree(src_h, dst_h)
    dS_t = dS_p.reshape(NW, HROWS * 128).T
    dD_t = dD_p.reshape(NW, HROWS * 128).T

    xs, ns, nd = _scale(feats, dS_t[:N], dD_t[:N])

    p1 = _aggregate(xs, src4, dst4)
    h1, t2 = _dense1(p1[0], p1[1], ns, nd, W1, b1.reshape(1, D), W2)

    p2 = _aggregate(t2, src4, dst4)
    h2 = _dense2(p2[0], p2[1], nd, b2.reshape(1, D))
    return (h1, h2)


# speedup vs baseline: 1.0274x; 1.0274x over previous
import jax
import jax.numpy as jnp
from jax import lax
from jax.experimental import pallas as pl
from jax.experimental.pallas import tpu as pltpu
from jax.experimental.pallas import tpu_sc as plsc

N = 10000
E = 320000
D = 128

NC = 2
NS = 16
NW = NC * NS
EPW = E // NW
CHUNK = 128
NGRP = 10
EPW_P = NGRP * 8 * CHUNK
PAD = EPW_P - EPW
ACC_ROWS = 10048
TRASH = 10040
HROWS = 80
NFL = 125

_MESH = plsc.VectorSubcoreMesh(
    core_axis_name="c", subcore_axis_name="s", num_cores=NC, num_subcores=NS
)
_SC_PARAMS = pltpu.CompilerParams(needs_layout_passes=False)


def _degree_body(src_hbm, dst_hbm, out_s, out_d, sidx, didx, ps, pd):
    cid = lax.axis_index("c")
    sid = lax.axis_index("s")
    wid = cid * NS + sid
    pltpu.sync_copy(src_hbm.at[wid], sidx)
    pltpu.sync_copy(dst_hbm.at[wid], didx)

    zeros = jnp.zeros((16,), jnp.float32)

    @pl.loop(0, HROWS)
    def _zero(i):
        for j in range(8):
            ps[i, pl.ds(j * 16, 16)] = zeros
            pd[i, pl.ds(j * 16, 16)] = zeros

    ones = jnp.ones((16,), jnp.float32)
    m127 = jnp.full((16,), 127, jnp.int32)

    @pl.loop(0, HROWS)
    def _hist(i):
        for j in range(8):
            s = sidx[i, pl.ds(j * 16, 16)]
            d = didx[i, pl.ds(j * 16, 16)]
            plsc.addupdate_scatter(ps, [s >> 7, s & m127], ones)
            plsc.addupdate_scatter(pd, [d >> 7, d & m127], ones)

    pltpu.sync_copy(ps, out_s.at[wid])
    pltpu.sync_copy(pd, out_d.at[wid])


_degree = pl.kernel(
    _degree_body,
    out_type=(
        jax.ShapeDtypeStruct((NW, HROWS, 128), jnp.float32),
        jax.ShapeDtypeStruct((NW, HROWS, 128), jnp.float32),
    ),
    mesh=_MESH,
    compiler_params=_SC_PARAMS,
    scratch_types=[
        pltpu.VMEM((HROWS, 128), jnp.int32),
        pltpu.VMEM((HROWS, 128), jnp.int32),
        pltpu.VMEM((HROWS, 128), jnp.float32),
        pltpu.VMEM((HROWS, 128), jnp.float32),
    ],
)


def _agg_body(table, src_hbm, dst_hbm, out, shared, si, di0, di1,
              rows0, rows1, semg0, semg1, semi0, semi1):
    cid = lax.axis_index("c")
    sid = lax.axis_index("s")
    wid = cid * NS + sid

    @pl.loop(0, NGRP)
    def _stg(g):
        pltpu.sync_copy(src_hbm.at[wid, g], si.at[pl.ds(g * 8, 8)])

    zeros = jnp.zeros((16,), jnp.float32)

    @pl.loop(0, CHUNK)
    def _z(i):
        for j in range(8):
            rows0[i, pl.ds(j * 16, 16)] = zeros

    @pl.loop(0, 8)
    def _zs(m):
        ch = sid + m * NS

        @pl.when(ch * 80 < ACC_ROWS)
        def _():
            pltpu.sync_copy(rows0.at[pl.ds(0, 80)], shared.at[pl.ds(ch * 80, 80)])

    plsc.subcore_barrier()

    bufs = (rows0, rows1)
    semg = (semg0, semg1)

    def startG(b, c):
        pltpu.async_copy(table.at[si.at[c]], bufs[b], semg[b])

    def waitG(b, c):
        pltpu.make_async_copy(table.at[si.at[c]], bufs[b], semg[b]).wait()

    pltpu.async_copy(dst_hbm.at[wid, 0], di0, semi0)
    startG(0, 0)

    @pl.loop(0, 5)
    def _pair(it):
        c0 = it * 16
        for r2 in range(16):
            c = c0 + r2
            b = r2 % 2
            ob = 1 - b
            dcur = di0 if r2 < 8 else di1
            ridx = r2 % 8
            if r2 == 0:
                pltpu.make_async_copy(dst_hbm.at[wid, 0], di0, semi0).wait()
            if r2 == 8:
                pltpu.make_async_copy(dst_hbm.at[wid, 0], di1, semi1).wait()

            @pl.when(c + 1 < NGRP * 8)
            def _():
                startG(ob, c + 1)

            waitG(b, c)
            pltpu.sync_copy(bufs[b], shared.at[dcur.at[ridx]], add=True)

            if r2 == 2:
                pltpu.async_copy(dst_hbm.at[wid, 2 * it + 1], di1, semi1)
            if r2 == 10:
                @pl.when(2 * it + 2 < NGRP)
                def _():
                    pltpu.async_copy(dst_hbm.at[wid, 2 * it + 2], di0, semi0)

    plsc.subcore_barrier()

    @pl.loop(0, 8)
    def _fl(m):
        ch = sid + m * NS

        @pl.when(ch < NFL)
        def _():
            pltpu.sync_copy(shared.at[pl.ds(ch * 80, 80)], rows0.at[pl.ds(0, 80)])
            pltpu.sync_copy(rows0.at[pl.ds(0, 80)], out.at[cid, pl.ds(ch * 80, 80)])


_aggregate = pl.kernel(
    _agg_body,
    out_type=jax.ShapeDtypeStruct((NC, N, D), jnp.float32),
    mesh=_MESH,
    compiler_params=_SC_PARAMS,
    scratch_types=[
        pltpu.VMEM_SHARED((ACC_ROWS, D), jnp.float32),
        pltpu.VMEM((NGRP * 8, CHUNK), jnp.int32),
        pltpu.VMEM((8, CHUNK), jnp.int32),
        pltpu.VMEM((8, CHUNK), jnp.int32),
        pltpu.VMEM((CHUNK, D), jnp.float32),
        pltpu.VMEM((CHUNK, D), jnp.float32),
        pltpu.SemaphoreType.DMA,
        pltpu.SemaphoreType.DMA,
        pltpu.SemaphoreType.DMA,
        pltpu.SemaphoreType.DMA,
    ],
)


_RB = 1000
_NB = N // _RB
_SRC_PAD_COUNT = float(NW * PAD)


def _scale_body(x_ref, ds_ref, dd_ref, xs_ref, ns_ref, nd_ref):
    i = pl.program_id(0)
    ds = jnp.sum(ds_ref[...], axis=1, keepdims=True)
    dd = jnp.sum(dd_ref[...], axis=1, keepdims=True)
    row0 = (lax.broadcasted_iota(jnp.int32, (_RB, 1), 0) == 0) & (i == 0)
    ds = ds - jnp.where(row0, _SRC_PAD_COUNT, 0.0)
    ns = lax.rsqrt(jnp.maximum(ds, 1.0))
    nd = lax.rsqrt(jnp.maximum(dd, 1.0))
    xs_ref[...] = x_ref[...] * ns
    ns_ref[...] = ns
    nd_ref[...] = nd


def _scale(x, ds_t, dd_t):
    return pl.pallas_call(
        _scale_body,
        grid=(_NB,),
        in_specs=[
            pl.BlockSpec((_RB, D), lambda i: (i, 0)),
            pl.BlockSpec((_RB, NW), lambda i: (i, 0)),
            pl.BlockSpec((_RB, NW), lambda i: (i, 0)),
        ],
        out_specs=[
            pl.BlockSpec((_RB, D), lambda i: (i, 0)),
            pl.BlockSpec((_RB, 1), lambda i: (i, 0)),
            pl.BlockSpec((_RB, 1), lambda i: (i, 0)),
        ],
        out_shape=[
            jax.ShapeDtypeStruct((N, D), jnp.float32),
            jax.ShapeDtypeStruct((N, 1), jnp.float32),
            jax.ShapeDtypeStruct((N, 1), jnp.float32),
        ],
    )(x, ds_t, dd_t)


def _dense1_body(p1a, p1b, ns, nd, w1, b1, w2, h1_o, t2_o):
    agg = (p1a[...] + p1b[...]) * nd[...]
    h1 = jnp.maximum(
        jnp.dot(agg, w1[...], preferred_element_type=jnp.float32) + b1[...], 0.0
    )
    h1_o[...] = h1
    t2_o[...] = ns[...] * jnp.dot(h1, w2[...], preferred_element_type=jnp.float32)


def _dense1(p1a, p1b, ns, nd, w1, b1, w2):
    return pl.pallas_call(
        _dense1_body,
        grid=(_NB,),
        in_specs=[
            pl.BlockSpec((_RB, D), lambda i: (i, 0)),
            pl.BlockSpec((_RB, D), lambda i: (i, 0)),
            pl.BlockSpec((_RB, 1), lambda i: (i, 0)),
            pl.BlockSpec((_RB, 1), lambda i: (i, 0)),
            pl.BlockSpec((D, D), lambda i: (0, 0)),
            pl.BlockSpec((1, D), lambda i: (0, 0)),
            pl.BlockSpec((D, D), lambda i: (0, 0)),
        ],
        out_specs=[
            pl.BlockSpec((_RB, D), lambda i: (i, 0)),
            pl.BlockSpec((_RB, D), lambda i: (i, 0)),
        ],
        out_shape=[
            jax.ShapeDtypeStruct((N, D), jnp.float32),
            jax.ShapeDtypeStruct((N, D), jnp.float32),
        ],
    )(p1a, p1b, ns, nd, w1, b1, w2)


def _dense2_body(p2a, p2b, nd, b2, h2_o):
    h2_o[...] = (p2a[...] + p2b[...]) * nd[...] + b2[...]


def _dense2(p2a, p2b, nd, b2):
    return pl.pallas_call(
        _dense2_body,
        grid=(_NB,),
        in_specs=[
            pl.BlockSpec((_RB, D), lambda i: (i, 0)),
            pl.BlockSpec((_RB, D), lambda i: (i, 0)),
            pl.BlockSpec((_RB, 1), lambda i: (i, 0)),
            pl.BlockSpec((1, D), lambda i: (0, 0)),
        ],
        out_specs=pl.BlockSpec((_RB, D), lambda i: (i, 0)),
        out_shape=jax.ShapeDtypeStruct((N, D), jnp.float32),
    )(p2a, p2b, nd, b2)


@jax.jit
def kernel(feats, edge_index, W1, b1, W2, b2):
    src = edge_index[0].reshape(NW, EPW)
    dst = edge_index[1].reshape(NW, EPW)
    src_p = jnp.pad(src, ((0, 0), (0, PAD)), constant_values=0)
    dst_p = jnp.pad(dst, ((0, 0), (0, PAD)), constant_values=TRASH)
    src4 = src_p.reshape(NW, NGRP, 8, CHUNK)
    dst4 = dst_p.reshape(NW, NGRP, 8, CHUNK)
    src_h = src_p.reshape(NW, HROWS, 128)
    dst_h = dst_p.reshape(NW, HROWS, 128)

    dS_p, dD_p = _degree(src_h, dst_h)
    dS_t = dS_p.reshape(NW, HROWS * 128).T
    dD_t = dD_p.reshape(NW, HROWS * 128).T

    xs, ns, nd = _scale(feats, dS_t[:N], dD_t[:N])

    p1 = _aggregate(xs, src4, dst4)
    h1, t2 = _dense1(p1[0], p1[1], ns, nd, W1, b1.reshape(1, D), W2)

    p2 = _aggregate(t2, src4, dst4)
    h2 = _dense2(p2[0], p2[1], nd, b2.reshape(1, D))
    return (h1, h2)


# speedup vs baseline: 1.7490x; 1.7024x over previous
import jax
import jax.numpy as jnp
from jax import lax
from jax.experimental import pallas as pl
from jax.experimental.pallas import tpu as pltpu
from jax.experimental.pallas import tpu_sc as plsc

N = 10000
E = 320000
D = 128

NC = 2
NS = 16
NW = NC * NS
EPW = E // NW
CHUNK = 128
NGRP = 10
EPW_P = NGRP * 8 * CHUNK
PAD = EPW_P - EPW
ACC_ROWS = 10048
TRASH = 10040
HROWS = 80
NFL = 125

_MESH = plsc.VectorSubcoreMesh(
    core_axis_name="c", subcore_axis_name="s", num_cores=NC, num_subcores=NS
)
_SC_PARAMS = pltpu.CompilerParams(needs_layout_passes=False)


def _degree_body(src_hbm, dst_hbm, out_s, out_d, sidx, didx, ps, pd):
    cid = lax.axis_index("c")
    sid = lax.axis_index("s")
    wid = cid * NS + sid
    pltpu.sync_copy(src_hbm.at[wid], sidx)
    pltpu.sync_copy(dst_hbm.at[wid], didx)

    zeros = jnp.zeros((16,), jnp.float32)

    @pl.loop(0, HROWS)
    def _zero(i):
        for j in range(8):
            ps[i, pl.ds(j * 16, 16)] = zeros
            pd[i, pl.ds(j * 16, 16)] = zeros

    ones = jnp.ones((16,), jnp.float32)
    m127 = jnp.full((16,), 127, jnp.int32)

    @pl.loop(0, HROWS)
    def _hist(i):
        for j in range(8):
            s = sidx[i, pl.ds(j * 16, 16)]
            d = didx[i, pl.ds(j * 16, 16)]
            plsc.addupdate_scatter(ps, [s >> 7, s & m127], ones)
            plsc.addupdate_scatter(pd, [d >> 7, d & m127], ones)

    pltpu.sync_copy(ps, out_s.at[wid])
    pltpu.sync_copy(pd, out_d.at[wid])


_degree = pl.kernel(
    _degree_body,
    out_type=(
        jax.ShapeDtypeStruct((NW, HROWS, 128), jnp.float32),
        jax.ShapeDtypeStruct((NW, HROWS, 128), jnp.float32),
    ),
    mesh=_MESH,
    compiler_params=_SC_PARAMS,
    scratch_types=[
        pltpu.VMEM((HROWS, 128), jnp.int32),
        pltpu.VMEM((HROWS, 128), jnp.int32),
        pltpu.VMEM((HROWS, 128), jnp.float32),
        pltpu.VMEM((HROWS, 128), jnp.float32),
    ],
)


def _agg_body(table, src_hbm, dst_hbm, out, shared, si0, si1, di0, di1,
              rows0, rows1, semg0, semg1, semis1, semid1, semis0, semid0):
    cid = lax.axis_index("c")
    sid = lax.axis_index("s")
    wid = cid * NS + sid

    zeros = jnp.zeros((16,), jnp.float32)

    @pl.loop(0, CHUNK)
    def _z(i):
        for j in range(8):
            rows0[i, pl.ds(j * 16, 16)] = zeros

    @pl.loop(0, 8)
    def _zs(m):
        ch = sid + m * NS

        @pl.when(ch * 80 < ACC_ROWS)
        def _():
            pltpu.sync_copy(rows0.at[pl.ds(0, 80)], shared.at[pl.ds(ch * 80, 80)])

    plsc.subcore_barrier()

    bufs = (rows0, rows1)
    semg = (semg0, semg1)
    sis = (si0, si1)

    def startG(b, sibuf, row):
        pltpu.async_copy(table.at[sibuf.at[row]], bufs[b], semg[b])

    def waitG(b, sibuf, row):
        pltpu.make_async_copy(table.at[sibuf.at[row]], bufs[b], semg[b]).wait()

    pltpu.sync_copy(src_hbm.at[wid, 0], si0)
    pltpu.async_copy(dst_hbm.at[wid, 0], di0, semid0)
    startG(0, si0, 0)

    @pl.loop(0, 5)
    def _pair(it):
        for r2 in range(16):
            c_next = it * 16 + r2 + 1
            b = r2 % 2
            ob = 1 - b
            dcur = di0 if r2 < 8 else di1
            scur = si0 if r2 < 8 else si1
            ridx = r2 % 8
            if r2 == 0:
                pltpu.make_async_copy(dst_hbm.at[wid, 0], di0, semid0).wait()
            if r2 == 1:
                pltpu.async_copy(src_hbm.at[wid, 2 * it + 1], si1, semis1)
                pltpu.async_copy(dst_hbm.at[wid, 2 * it + 1], di1, semid1)
            if r2 == 6:
                pltpu.make_async_copy(src_hbm.at[wid, 0], si1, semis1).wait()
            if r2 == 8:
                pltpu.make_async_copy(dst_hbm.at[wid, 0], di1, semid1).wait()
            if r2 == 9:
                @pl.when(2 * it + 2 < NGRP)
                def _():
                    pltpu.async_copy(src_hbm.at[wid, 2 * it + 2], si0, semis0)
                    pltpu.async_copy(dst_hbm.at[wid, 2 * it + 2], di0, semid0)

            if r2 < 7:
                startG(ob, scur, ridx + 1)
            elif r2 == 7:
                startG(ob, si1, 0)
            elif r2 < 15:
                startG(ob, si1, ridx + 1)
            else:
                @pl.when(c_next < NGRP * 8)
                def _():
                    pltpu.make_async_copy(src_hbm.at[wid, 0], si0, semis0).wait()
                    startG(ob, si0, 0)

            waitG(b, scur, ridx)
            pltpu.sync_copy(bufs[b], shared.at[dcur.at[ridx]], add=True)

    plsc.subcore_barrier()

    @pl.loop(0, 8)
    def _fl(m):
        ch = sid + m * NS

        @pl.when(ch < NFL)
        def _():
            pltpu.sync_copy(shared.at[pl.ds(ch * 80, 80)], rows0.at[pl.ds(0, 80)])
            pltpu.sync_copy(rows0.at[pl.ds(0, 80)], out.at[cid, pl.ds(ch * 80, 80)])


_aggregate = pl.kernel(
    _agg_body,
    out_type=jax.ShapeDtypeStruct((NC, N, D), jnp.float32),
    mesh=_MESH,
    compiler_params=_SC_PARAMS,
    scratch_types=[
        pltpu.VMEM_SHARED((ACC_ROWS, D), jnp.float32),
        pltpu.VMEM((8, CHUNK), jnp.int32),
        pltpu.VMEM((8, CHUNK), jnp.int32),
        pltpu.VMEM((8, CHUNK), jnp.int32),
        pltpu.VMEM((8, CHUNK), jnp.int32),
        pltpu.VMEM((CHUNK, D), jnp.float32),
        pltpu.VMEM((CHUNK, D), jnp.float32),
        pltpu.SemaphoreType.DMA,
        pltpu.SemaphoreType.DMA,
        pltpu.SemaphoreType.DMA,
        pltpu.SemaphoreType.DMA,
        pltpu.SemaphoreType.DMA,
        pltpu.SemaphoreType.DMA,
    ],
)


_RB = 1000
_NB = N // _RB
_SRC_PAD_COUNT = float(NW * PAD)


def _scale_body(x_ref, ds_ref, dd_ref, xs_ref, ns_ref, nd_ref):
    i = pl.program_id(0)
    ds = jnp.sum(ds_ref[...], axis=1, keepdims=True)
    dd = jnp.sum(dd_ref[...], axis=1, keepdims=True)
    row0 = (lax.broadcasted_iota(jnp.int32, (_RB, 1), 0) == 0) & (i == 0)
    ds = ds - jnp.where(row0, _SRC_PAD_COUNT, 0.0)
    ns = lax.rsqrt(jnp.maximum(ds, 1.0))
    nd = lax.rsqrt(jnp.maximum(dd, 1.0))
    xs_ref[...] = x_ref[...] * ns
    ns_ref[...] = ns
    nd_ref[...] = nd


def _scale(x, ds_t, dd_t):
    return pl.pallas_call(
        _scale_body,
        grid=(_NB,),
        in_specs=[
            pl.BlockSpec((_RB, D), lambda i: (i, 0)),
            pl.BlockSpec((_RB, NW), lambda i: (i, 0)),
            pl.BlockSpec((_RB, NW), lambda i: (i, 0)),
        ],
        out_specs=[
            pl.BlockSpec((_RB, D), lambda i: (i, 0)),
            pl.BlockSpec((_RB, 1), lambda i: (i, 0)),
            pl.BlockSpec((_RB, 1), lambda i: (i, 0)),
        ],
        out_shape=[
            jax.ShapeDtypeStruct((N, D), jnp.float32),
            jax.ShapeDtypeStruct((N, 1), jnp.float32),
            jax.ShapeDtypeStruct((N, 1), jnp.float32),
        ],
    )(x, ds_t, dd_t)


def _dense1_body(p1a, p1b, ns, nd, w1, b1, w2, h1_o, t2_o):
    agg = (p1a[...] + p1b[...]) * nd[...]
    h1 = jnp.maximum(
        jnp.dot(agg, w1[...], preferred_element_type=jnp.float32) + b1[...], 0.0
    )
    h1_o[...] = h1
    t2_o[...] = ns[...] * jnp.dot(h1, w2[...], preferred_element_type=jnp.float32)


def _dense1(p1a, p1b, ns, nd, w1, b1, w2):
    return pl.pallas_call(
        _dense1_body,
        grid=(_NB,),
        in_specs=[
            pl.BlockSpec((_RB, D), lambda i: (i, 0)),
            pl.BlockSpec((_RB, D), lambda i: (i, 0)),
            pl.BlockSpec((_RB, 1), lambda i: (i, 0)),
            pl.BlockSpec((_RB, 1), lambda i: (i, 0)),
            pl.BlockSpec((D, D), lambda i: (0, 0)),
            pl.BlockSpec((1, D), lambda i: (0, 0)),
            pl.BlockSpec((D, D), lambda i: (0, 0)),
        ],
        out_specs=[
            pl.BlockSpec((_RB, D), lambda i: (i, 0)),
            pl.BlockSpec((_RB, D), lambda i: (i, 0)),
        ],
        out_shape=[
            jax.ShapeDtypeStruct((N, D), jnp.float32),
            jax.ShapeDtypeStruct((N, D), jnp.float32),
        ],
    )(p1a, p1b, ns, nd, w1, b1, w2)


def _dense2_body(p2a, p2b, nd, b2, h2_o):
    h2_o[...] = (p2a[...] + p2b[...]) * nd[...] + b2[...]


def _dense2(p2a, p2b, nd, b2):
    return pl.pallas_call(
        _dense2_body,
        grid=(_NB,),
        in_specs=[
            pl.BlockSpec((_RB, D), lambda i: (i, 0)),
            pl.BlockSpec((_RB, D), lambda i: (i, 0)),
            pl.BlockSpec((_RB, 1), lambda i: (i, 0)),
            pl.BlockSpec((1, D), lambda i: (0, 0)),
        ],
        out_specs=pl.BlockSpec((_RB, D), lambda i: (i, 0)),
        out_shape=jax.ShapeDtypeStruct((N, D), jnp.float32),
    )(p2a, p2b, nd, b2)


@jax.jit
def kernel(feats, edge_index, W1, b1, W2, b2):
    src = edge_index[0].reshape(NW, EPW)
    dst = edge_index[1].reshape(NW, EPW)
    src_p = jnp.pad(src, ((0, 0), (0, PAD)), constant_values=0)
    dst_p = jnp.pad(dst, ((0, 0), (0, PAD)), constant_values=TRASH)
    src4 = src_p.reshape(NW, NGRP, 8, CHUNK)
    dst4 = dst_p.reshape(NW, NGRP, 8, CHUNK)
    src_h = src_p.reshape(NW, HROWS, 128)
    dst_h = dst_p.reshape(NW, HROWS, 128)

    dS_p, dD_p = _degree(src_h, dst_h)
    dS_t = dS_p.reshape(NW, HROWS * 128).T
    dD_t = dD_p.reshape(NW, HROWS * 128).T

    xs, ns, nd = _scale(feats, dS_t[:N], dD_t[:N])

    p1 = _aggregate(xs, src4, dst4)
    h1, t2 = _dense1(p1[0], p1[1], ns, nd, W1, b1.reshape(1, D), W2)

    p2 = _aggregate(t2, src4, dst4)
    h2 = _dense2(p2[0], p2[1], nd, b2.reshape(1, D))
    return (h1, h2)


# speedup vs baseline: 1.7492x; 1.0001x over previous
import jax
import jax.numpy as jnp
from jax import lax
from jax.experimental import pallas as pl
from jax.experimental.pallas import tpu as pltpu
from jax.experimental.pallas import tpu_sc as plsc

N = 10000
E = 320000
D = 128

NC = 2
NS = 16
NW = NC * NS
EPW = E // NW
CHUNK = 128
NGRP = 10
EPW_P = NGRP * 8 * CHUNK
PAD = EPW_P - EPW
ACC_ROWS = 10048
TRASH = 10040
HROWS = 80
NFL = 125

_MESH = plsc.VectorSubcoreMesh(
    core_axis_name="c", subcore_axis_name="s", num_cores=NC, num_subcores=NS
)
_SC_PARAMS = pltpu.CompilerParams(needs_layout_passes=False)


def _degree_body(src_hbm, dst_hbm, out_s, out_d, sidx, didx, ps, pd):
    cid = lax.axis_index("c")
    sid = lax.axis_index("s")
    wid = cid * NS + sid
    pltpu.sync_copy(src_hbm.at[wid], sidx)
    pltpu.sync_copy(dst_hbm.at[wid], didx)

    zeros = jnp.zeros((16,), jnp.float32)

    @pl.loop(0, HROWS)
    def _zero(i):
        for j in range(8):
            ps[i, pl.ds(j * 16, 16)] = zeros
            pd[i, pl.ds(j * 16, 16)] = zeros

    ones = jnp.ones((16,), jnp.float32)
    m127 = jnp.full((16,), 127, jnp.int32)

    @pl.loop(0, HROWS)
    def _hist(i):
        for j in range(8):
            s = sidx[i, pl.ds(j * 16, 16)]
            d = didx[i, pl.ds(j * 16, 16)]
            plsc.addupdate_scatter(ps, [s >> 7, s & m127], ones)
            plsc.addupdate_scatter(pd, [d >> 7, d & m127], ones)

    pltpu.sync_copy(ps, out_s.at[wid])
    pltpu.sync_copy(pd, out_d.at[wid])


_degree = pl.kernel(
    _degree_body,
    out_type=(
        jax.ShapeDtypeStruct((NW, HROWS, 128), jnp.float32),
        jax.ShapeDtypeStruct((NW, HROWS, 128), jnp.float32),
    ),
    mesh=_MESH,
    compiler_params=_SC_PARAMS,
    scratch_types=[
        pltpu.VMEM((HROWS, 128), jnp.int32),
        pltpu.VMEM((HROWS, 128), jnp.int32),
        pltpu.VMEM((HROWS, 128), jnp.float32),
        pltpu.VMEM((HROWS, 128), jnp.float32),
    ],
)


def _agg_body(table, src_hbm, dst_hbm, out, shared, si0, si1, di0, di1,
              rows0, rows1, semg0, semg1, semis1, semid1, semis0, semid0,
              semsc0, semsc1):
    cid = lax.axis_index("c")
    sid = lax.axis_index("s")
    wid = cid * NS + sid

    zeros = jnp.zeros((16,), jnp.float32)

    @pl.loop(0, CHUNK)
    def _z(i):
        for j in range(8):
            rows0[i, pl.ds(j * 16, 16)] = zeros

    @pl.loop(0, 8)
    def _zs(m):
        ch = sid + m * NS

        @pl.when(ch * 80 < ACC_ROWS)
        def _():
            pltpu.sync_copy(rows0.at[pl.ds(0, 80)], shared.at[pl.ds(ch * 80, 80)])

    plsc.subcore_barrier()

    bufs = (rows0, rows1)
    semg = (semg0, semg1)
    semsc = (semsc0, semsc1)

    def startG(b, sibuf, row):
        pltpu.async_copy(table.at[sibuf.at[row]], bufs[b], semg[b])

    def waitG(b, sibuf, row):
        pltpu.make_async_copy(table.at[sibuf.at[row]], bufs[b], semg[b]).wait()

    def startS(b, dbuf, row):
        pltpu.async_copy(bufs[b], shared.at[dbuf.at[row]], semsc[b], add=True)

    def waitS(b, dbuf, row):
        pltpu.make_async_copy(bufs[b], shared.at[dbuf.at[row]], semsc[b]).wait()

    pltpu.sync_copy(src_hbm.at[wid, 0], si0)
    pltpu.async_copy(dst_hbm.at[wid, 0], di0, semid0)
    startG(0, si0, 0)

    @pl.loop(0, 5)
    def _pair(it):
        for r2 in range(16):
            c_next = it * 16 + r2 + 1
            b = r2 % 2
            ob = 1 - b
            dcur = di0 if r2 < 8 else di1
            scur = si0 if r2 < 8 else si1
            ridx = r2 % 8
            if r2 == 0:
                pltpu.make_async_copy(dst_hbm.at[wid, 0], di0, semid0).wait()
            if r2 == 1:
                pltpu.async_copy(src_hbm.at[wid, 2 * it + 1], si1, semis1)
                pltpu.async_copy(dst_hbm.at[wid, 2 * it + 1], di1, semid1)
            if r2 == 6:
                pltpu.make_async_copy(src_hbm.at[wid, 0], si1, semis1).wait()
            if r2 == 8:
                pltpu.make_async_copy(dst_hbm.at[wid, 0], di1, semid1).wait()
            if r2 == 9:
                @pl.when(2 * it + 2 < NGRP)
                def _():
                    pltpu.async_copy(src_hbm.at[wid, 2 * it + 2], si0, semis0)
                    pltpu.async_copy(dst_hbm.at[wid, 2 * it + 2], di0, semid0)

            if r2 == 0:
                @pl.when(it > 0)
                def _():
                    waitS(ob, di1, 7)
            else:
                pdcur = di0 if (r2 - 1) < 8 else di1
                waitS(ob, pdcur, (r2 - 1) % 8)

            if r2 < 7:
                startG(ob, scur, ridx + 1)
            elif r2 == 7:
                startG(ob, si1, 0)
            elif r2 < 15:
                startG(ob, si1, ridx + 1)
            else:
                @pl.when(c_next < NGRP * 8)
                def _():
                    pltpu.make_async_copy(src_hbm.at[wid, 0], si0, semis0).wait()
                    startG(ob, si0, 0)

            waitG(b, scur, ridx)
            startS(b, dcur, ridx)

    waitS(1, di1, 7)
    plsc.subcore_barrier()

    @pl.loop(0, 8)
    def _fl(m):
        ch = sid + m * NS

        @pl.when(ch < NFL)
        def _():
            pltpu.sync_copy(shared.at[pl.ds(ch * 80, 80)], rows0.at[pl.ds(0, 80)])
            pltpu.sync_copy(rows0.at[pl.ds(0, 80)], out.at[cid, pl.ds(ch * 80, 80)])


_aggregate = pl.kernel(
    _agg_body,
    out_type=jax.ShapeDtypeStruct((NC, N, D), jnp.float32),
    mesh=_MESH,
    compiler_params=_SC_PARAMS,
    scratch_types=[
        pltpu.VMEM_SHARED((ACC_ROWS, D), jnp.float32),
        pltpu.VMEM((8, CHUNK), jnp.int32),
        pltpu.VMEM((8, CHUNK), jnp.int32),
        pltpu.VMEM((8, CHUNK), jnp.int32),
        pltpu.VMEM((8, CHUNK), jnp.int32),
        pltpu.VMEM((CHUNK, D), jnp.float32),
        pltpu.VMEM((CHUNK, D), jnp.float32),
        pltpu.SemaphoreType.DMA,
        pltpu.SemaphoreType.DMA,
        pltpu.SemaphoreType.DMA,
        pltpu.SemaphoreType.DMA,
        pltpu.SemaphoreType.DMA,
        pltpu.SemaphoreType.DMA,
        pltpu.SemaphoreType.DMA,
        pltpu.SemaphoreType.DMA,
    ],
)


_RB = 1000
_NB = N // _RB
_SRC_PAD_COUNT = float(NW * PAD)


def _scale_body(x_ref, ds_ref, dd_ref, xs_ref, ns_ref, nd_ref):
    i = pl.program_id(0)
    ds = jnp.sum(ds_ref[...], axis=1, keepdims=True)
    dd = jnp.sum(dd_ref[...], axis=1, keepdims=True)
    row0 = (lax.broadcasted_iota(jnp.int32, (_RB, 1), 0) == 0) & (i == 0)
    ds = ds - jnp.where(row0, _SRC_PAD_COUNT, 0.0)
    ns = lax.rsqrt(jnp.maximum(ds, 1.0))
    nd = lax.rsqrt(jnp.maximum(dd, 1.0))
    xs_ref[...] = x_ref[...] * ns
    ns_ref[...] = ns
    nd_ref[...] = nd


def _scale(x, ds_t, dd_t):
    return pl.pallas_call(
        _scale_body,
        grid=(_NB,),
        in_specs=[
            pl.BlockSpec((_RB, D), lambda i: (i, 0)),
            pl.BlockSpec((_RB, NW), lambda i: (i, 0)),
            pl.BlockSpec((_RB, NW), lambda i: (i, 0)),
        ],
        out_specs=[
            pl.BlockSpec((_RB, D), lambda i: (i, 0)),
            pl.BlockSpec((_RB, 1), lambda i: (i, 0)),
            pl.BlockSpec((_RB, 1), lambda i: (i, 0)),
        ],
        out_shape=[
            jax.ShapeDtypeStruct((N, D), jnp.float32),
            jax.ShapeDtypeStruct((N, 1), jnp.float32),
            jax.ShapeDtypeStruct((N, 1), jnp.float32),
        ],
    )(x, ds_t, dd_t)


def _dense1_body(p1a, p1b, ns, nd, w1, b1, w2, h1_o, t2_o):
    agg = (p1a[...] + p1b[...]) * nd[...]
    h1 = jnp.maximum(
        jnp.dot(agg, w1[...], preferred_element_type=jnp.float32) + b1[...], 0.0
    )
    h1_o[...] = h1
    t2_o[...] = ns[...] * jnp.dot(h1, w2[...], preferred_element_type=jnp.float32)


def _dense1(p1a, p1b, ns, nd, w1, b1, w2):
    return pl.pallas_call(
        _dense1_body,
        grid=(_NB,),
        in_specs=[
            pl.BlockSpec((_RB, D), lambda i: (i, 0)),
            pl.BlockSpec((_RB, D), lambda i: (i, 0)),
            pl.BlockSpec((_RB, 1), lambda i: (i, 0)),
            pl.BlockSpec((_RB, 1), lambda i: (i, 0)),
            pl.BlockSpec((D, D), lambda i: (0, 0)),
            pl.BlockSpec((1, D), lambda i: (0, 0)),
            pl.BlockSpec((D, D), lambda i: (0, 0)),
        ],
        out_specs=[
            pl.BlockSpec((_RB, D), lambda i: (i, 0)),
            pl.BlockSpec((_RB, D), lambda i: (i, 0)),
        ],
        out_shape=[
            jax.ShapeDtypeStruct((N, D), jnp.float32),
            jax.ShapeDtypeStruct((N, D), jnp.float32),
        ],
    )(p1a, p1b, ns, nd, w1, b1, w2)


def _dense2_body(p2a, p2b, nd, b2, h2_o):
    h2_o[...] = (p2a[...] + p2b[...]) * nd[...] + b2[...]


def _dense2(p2a, p2b, nd, b2):
    return pl.pallas_call(
        _dense2_body,
        grid=(_NB,),
        in_specs=[
            pl.BlockSpec((_RB, D), lambda i: (i, 0)),
            pl.BlockSpec((_RB, D), lambda i: (i, 0)),
            pl.BlockSpec((_RB, 1), lambda i: (i, 0)),
            pl.BlockSpec((1, D), lambda i: (0, 0)),
        ],
        out_specs=pl.BlockSpec((_RB, D), lambda i: (i, 0)),
        out_shape=jax.ShapeDtypeStruct((N, D), jnp.float32),
    )(p2a, p2b, nd, b2)


@jax.jit
def kernel(feats, edge_index, W1, b1, W2, b2):
    src = edge_index[0].reshape(NW, EPW)
    dst = edge_index[1].reshape(NW, EPW)
    src_p = jnp.pad(src, ((0, 0), (0, PAD)), constant_values=0)
    dst_p = jnp.pad(dst, ((0, 0), (0, PAD)), constant_values=TRASH)
    src4 = src_p.reshape(NW, NGRP, 8, CHUNK)
    dst4 = dst_p.reshape(NW, NGRP, 8, CHUNK)
    src_h = src_p.reshape(NW, HROWS, 128)
    dst_h = dst_p.reshape(NW, HROWS, 128)

    dS_p, dD_p = _degree(src_h, dst_h)
    dS_t = dS_p.reshape(NW, HROWS * 128).T
    dD_t = dD_p.reshape(NW, HROWS * 128).T

    xs, ns, nd = _scale(feats, dS_t[:N], dD_t[:N])

    p1 = _aggregate(xs, src4, dst4)
    h1, t2 = _dense1(p1[0], p1[1], ns, nd, W1, b1.reshape(1, D), W2)

    p2 = _aggregate(t2, src4, dst4)
    h2 = _dense2(p2[0], p2[1], nd, b2.reshape(1, D))
    return (h1, h2)


# speedup vs baseline: 1.7766x; 1.0156x over previous
import jax
import jax.numpy as jnp
from jax import lax
from jax.experimental import pallas as pl
from jax.experimental.pallas import tpu as pltpu
from jax.experimental.pallas import tpu_sc as plsc

N = 10000
E = 320000
D = 128

NC = 2
NS = 16
NW = NC * NS
EPW = E // NW
CHUNK = 128
NGRP = 10
EPW_P = NGRP * 8 * CHUNK
PAD = EPW_P - EPW
ACC_ROWS = 10048
TRASH = 10040
HROWS = 80
NFL = 125

_MESH = plsc.VectorSubcoreMesh(
    core_axis_name="c", subcore_axis_name="s", num_cores=NC, num_subcores=NS
)
_SC_PARAMS = pltpu.CompilerParams(needs_layout_passes=False)


def _degree_body(src_hbm, dst_hbm, out_s, out_d, sidx, didx, ps, pd):
    cid = lax.axis_index("c")
    sid = lax.axis_index("s")
    wid = cid * NS + sid
    pltpu.sync_copy(src_hbm.at[wid], sidx)
    pltpu.sync_copy(dst_hbm.at[wid], didx)

    zeros = jnp.zeros((16,), jnp.float32)

    @pl.loop(0, HROWS)
    def _zero(i):
        for j in range(8):
            ps[i, pl.ds(j * 16, 16)] = zeros
            pd[i, pl.ds(j * 16, 16)] = zeros

    ones = jnp.ones((16,), jnp.float32)
    m127 = jnp.full((16,), 127, jnp.int32)

    @pl.loop(0, HROWS)
    def _hist(i):
        for j in range(8):
            s = sidx[i, pl.ds(j * 16, 16)]
            d = didx[i, pl.ds(j * 16, 16)]
            plsc.addupdate_scatter(ps, [s >> 7, s & m127], ones)
            plsc.addupdate_scatter(pd, [d >> 7, d & m127], ones)

    pltpu.sync_copy(ps, out_s.at[wid])
    pltpu.sync_copy(pd, out_d.at[wid])


_degree = pl.kernel(
    _degree_body,
    out_type=(
        jax.ShapeDtypeStruct((NW, HROWS, 128), jnp.float32),
        jax.ShapeDtypeStruct((NW, HROWS, 128), jnp.float32),
    ),
    mesh=_MESH,
    compiler_params=_SC_PARAMS,
    scratch_types=[
        pltpu.VMEM((HROWS, 128), jnp.int32),
        pltpu.VMEM((HROWS, 128), jnp.int32),
        pltpu.VMEM((HROWS, 128), jnp.float32),
        pltpu.VMEM((HROWS, 128), jnp.float32),
    ],
)


def _agg_body(table, src_hbm, dst_hbm, out, shared, si0, si1, di0, di1,
              rows0, rows1, semg0, semg1, semis1, semid1, semis0, semid0,
              semsc0, semsc1):
    cid = lax.axis_index("c")
    sid = lax.axis_index("s")
    wid = cid * NS + sid

    zeros = jnp.zeros((16,), jnp.float32)

    @pl.loop(0, CHUNK)
    def _z(i):
        for j in range(8):
            rows0[i, pl.ds(j * 16, 16)] = zeros

    @pl.loop(0, 8)
    def _zs(m):
        ch = sid + m * NS

        @pl.when(ch * 80 < ACC_ROWS)
        def _():
            pltpu.sync_copy(rows0.at[pl.ds(0, 80)], shared.at[pl.ds(ch * 80, 80)])

    plsc.subcore_barrier()

    bufs = (rows0, rows1)
    semg = (semg0, semg1)
    semsc = (semsc0, semsc1)

    def startG(b, sibuf, row):
        pltpu.async_copy(table.at[sibuf.at[row]], bufs[b], semg[b])

    def waitG(b, sibuf, row):
        pltpu.make_async_copy(table.at[sibuf.at[row]], bufs[b], semg[b]).wait()

    def startS(b, dbuf, row):
        pltpu.async_copy(bufs[b], shared.at[dbuf.at[row]], semsc[b], add=True)

    def waitS(b, dbuf, row):
        pltpu.make_async_copy(bufs[b], shared.at[dbuf.at[row]], semsc[b]).wait()

    pltpu.sync_copy(src_hbm.at[wid, 0], si0)
    pltpu.async_copy(dst_hbm.at[wid, 0], di0, semid0)
    startG(0, si0, 0)

    @pl.loop(0, 5)
    def _pair(it):
        for r2 in range(16):
            c_next = it * 16 + r2 + 1
            b = r2 % 2
            ob = 1 - b
            dcur = di0 if r2 < 8 else di1
            scur = si0 if r2 < 8 else si1
            ridx = r2 % 8
            if r2 == 0:
                pltpu.make_async_copy(dst_hbm.at[wid, 0], di0, semid0).wait()
            if r2 == 1:
                pltpu.async_copy(src_hbm.at[wid, 2 * it + 1], si1, semis1)
                pltpu.async_copy(dst_hbm.at[wid, 2 * it + 1], di1, semid1)
            if r2 == 6:
                pltpu.make_async_copy(src_hbm.at[wid, 0], si1, semis1).wait()
            if r2 == 8:
                pltpu.make_async_copy(dst_hbm.at[wid, 0], di1, semid1).wait()
            if r2 == 9:
                @pl.when(2 * it + 2 < NGRP)
                def _():
                    pltpu.async_copy(src_hbm.at[wid, 2 * it + 2], si0, semis0)
                    pltpu.async_copy(dst_hbm.at[wid, 2 * it + 2], di0, semid0)

            if r2 == 0:
                @pl.when(it > 0)
                def _():
                    waitS(ob, di1, 7)
            else:
                pdcur = di0 if (r2 - 1) < 8 else di1
                waitS(ob, pdcur, (r2 - 1) % 8)

            if r2 < 7:
                startG(ob, scur, ridx + 1)
            elif r2 == 7:
                startG(ob, si1, 0)
            elif r2 < 15:
                startG(ob, si1, ridx + 1)
            else:
                @pl.when(c_next < NGRP * 8)
                def _():
                    pltpu.make_async_copy(src_hbm.at[wid, 0], si0, semis0).wait()
                    startG(ob, si0, 0)

            waitG(b, scur, ridx)
            startS(b, dcur, ridx)

    waitS(1, di1, 7)
    plsc.subcore_barrier()

    @pl.loop(0, 8)
    def _fl(m):
        ch = sid + m * NS

        @pl.when(ch < NFL)
        def _():
            pltpu.sync_copy(shared.at[pl.ds(ch * 80, 80)], rows0.at[pl.ds(0, 80)])
            pltpu.sync_copy(rows0.at[pl.ds(0, 80)], out.at[cid, pl.ds(ch * 80, 80)])


_aggregate = pl.kernel(
    _agg_body,
    out_type=jax.ShapeDtypeStruct((NC, N, D), jnp.float32),
    mesh=_MESH,
    compiler_params=_SC_PARAMS,
    scratch_types=[
        pltpu.VMEM_SHARED((ACC_ROWS, D), jnp.float32),
        pltpu.VMEM((8, CHUNK), jnp.int32),
        pltpu.VMEM((8, CHUNK), jnp.int32),
        pltpu.VMEM((8, CHUNK), jnp.int32),
        pltpu.VMEM((8, CHUNK), jnp.int32),
        pltpu.VMEM((CHUNK, D), jnp.float32),
        pltpu.VMEM((CHUNK, D), jnp.float32),
        pltpu.SemaphoreType.DMA,
        pltpu.SemaphoreType.DMA,
        pltpu.SemaphoreType.DMA,
        pltpu.SemaphoreType.DMA,
        pltpu.SemaphoreType.DMA,
        pltpu.SemaphoreType.DMA,
        pltpu.SemaphoreType.DMA,
        pltpu.SemaphoreType.DMA,
    ],
)


_RB = 1000
_NB = N // _RB
_SRC_PAD_COUNT = float(NW * PAD)


def _scale_body(x_ref, ds_ref, dd_ref, xs_ref, ns_ref, nd_ref):
    i = pl.program_id(0)
    ds = jnp.sum(ds_ref[...], axis=1, keepdims=True)
    dd = jnp.sum(dd_ref[...], axis=1, keepdims=True)
    row0 = (lax.broadcasted_iota(jnp.int32, (_RB, 1), 0) == 0) & (i == 0)
    ds = ds - jnp.where(row0, _SRC_PAD_COUNT, 0.0)
    ns = lax.rsqrt(jnp.maximum(ds, 1.0))
    nd = lax.rsqrt(jnp.maximum(dd, 1.0))
    xs_ref[...] = x_ref[...] * ns
    ns_ref[...] = ns
    nd_ref[...] = nd


def _scale(x, ds_t, dd_t):
    return pl.pallas_call(
        _scale_body,
        grid=(_NB,),
        in_specs=[
            pl.BlockSpec((_RB, D), lambda i: (i, 0)),
            pl.BlockSpec((_RB, NW), lambda i: (i, 0)),
            pl.BlockSpec((_RB, NW), lambda i: (i, 0)),
        ],
        out_specs=[
            pl.BlockSpec((_RB, D), lambda i: (i, 0)),
            pl.BlockSpec((_RB, 1), lambda i: (i, 0)),
            pl.BlockSpec((_RB, 1), lambda i: (i, 0)),
        ],
        out_shape=[
            jax.ShapeDtypeStruct((N, D), jnp.float32),
            jax.ShapeDtypeStruct((N, 1), jnp.float32),
            jax.ShapeDtypeStruct((N, 1), jnp.float32),
        ],
    )(x, ds_t, dd_t)


def _dense1_body(p1a, p1b, ns, nd, w1, b1, w2, h1_o, t2_o):
    agg = (p1a[0] + p1b[0]) * nd[...]
    h1 = jnp.maximum(
        jnp.dot(agg, w1[...], preferred_element_type=jnp.float32) + b1[...], 0.0
    )
    h1_o[...] = h1
    t2_o[...] = ns[...] * jnp.dot(h1, w2[...], preferred_element_type=jnp.float32)


def _dense1(p1a, p1b, ns, nd, w1, b1, w2):
    return pl.pallas_call(
        _dense1_body,
        grid=(_NB,),
        in_specs=[
            pl.BlockSpec((1, _RB, D), lambda i: (0, i, 0)),
            pl.BlockSpec((1, _RB, D), lambda i: (1, i, 0)),
            pl.BlockSpec((_RB, 1), lambda i: (i, 0)),
            pl.BlockSpec((_RB, 1), lambda i: (i, 0)),
            pl.BlockSpec((D, D), lambda i: (0, 0)),
            pl.BlockSpec((1, D), lambda i: (0, 0)),
            pl.BlockSpec((D, D), lambda i: (0, 0)),
        ],
        out_specs=[
            pl.BlockSpec((_RB, D), lambda i: (i, 0)),
            pl.BlockSpec((_RB, D), lambda i: (i, 0)),
        ],
        out_shape=[
            jax.ShapeDtypeStruct((N, D), jnp.float32),
            jax.ShapeDtypeStruct((N, D), jnp.float32),
        ],
    )(p1a, p1b, ns, nd, w1, b1, w2)


def _dense2_body(p2a, p2b, nd, b2, h2_o):
    h2_o[...] = (p2a[0] + p2b[0]) * nd[...] + b2[...]


def _dense2(p2a, p2b, nd, b2):
    return pl.pallas_call(
        _dense2_body,
        grid=(_NB,),
        in_specs=[
            pl.BlockSpec((1, _RB, D), lambda i: (0, i, 0)),
            pl.BlockSpec((1, _RB, D), lambda i: (1, i, 0)),
            pl.BlockSpec((_RB, 1), lambda i: (i, 0)),
            pl.BlockSpec((1, D), lambda i: (0, 0)),
        ],
        out_specs=pl.BlockSpec((_RB, D), lambda i: (i, 0)),
        out_shape=jax.ShapeDtypeStruct((N, D), jnp.float32),
    )(p2a, p2b, nd, b2)


@jax.jit
def kernel(feats, edge_index, W1, b1, W2, b2):
    src = edge_index[0].reshape(NW, EPW)
    dst = edge_index[1].reshape(NW, EPW)
    src_p = jnp.pad(src, ((0, 0), (0, PAD)), constant_values=0)
    dst_p = jnp.pad(dst, ((0, 0), (0, PAD)), constant_values=TRASH)
    src4 = src_p.reshape(NW, NGRP, 8, CHUNK)
    dst4 = dst_p.reshape(NW, NGRP, 8, CHUNK)
    src_h = src_p.reshape(NW, HROWS, 128)
    dst_h = dst_p.reshape(NW, HROWS, 128)

    dS_p, dD_p = _degree(src_h, dst_h)
    dS_t = dS_p.reshape(NW, HROWS * 128).T
    dD_t = dD_p.reshape(NW, HROWS * 128).T

    xs, ns, nd = _scale(feats, dS_t[:N], dD_t[:N])

    p1 = _aggregate(xs, src4, dst4)
    h1, t2 = _dense1(p1, p1, ns, nd, W1, b1.reshape(1, D), W2)

    p2 = _aggregate(t2, src4, dst4)
    h2 = _dense2(p2, p2, nd, b2.reshape(1, D))
    return (h1, h2)


# speedup vs baseline: 1.8013x; 1.0139x over previous
import jax
import jax.numpy as jnp
from jax import lax
from jax.experimental import pallas as pl
from jax.experimental.pallas import tpu as pltpu
from jax.experimental.pallas import tpu_sc as plsc

N = 10000
E = 320000
D = 128

NC = 2
NS = 16
NW = NC * NS
EPW = E // NW
CHUNK = 128
NGRP = 10
EPW_P = NGRP * 8 * CHUNK
PAD = EPW_P - EPW
ACC_ROWS = 10048
TRASH = 10040
HROWS = 80
NFL = 125

_MESH = plsc.VectorSubcoreMesh(
    core_axis_name="c", subcore_axis_name="s", num_cores=NC, num_subcores=NS
)
_SC_PARAMS = pltpu.CompilerParams(needs_layout_passes=False)


def _degree_body(src_hbm, dst_hbm, out_s, out_d, sidx, didx, ps, pd):
    cid = lax.axis_index("c")
    sid = lax.axis_index("s")
    wid = cid * NS + sid
    pltpu.sync_copy(src_hbm.at[wid], sidx)
    pltpu.sync_copy(dst_hbm.at[wid], didx)

    zeros = jnp.zeros((16,), jnp.float32)

    @pl.loop(0, HROWS)
    def _zero(i):
        for j in range(8):
            ps[i, pl.ds(j * 16, 16)] = zeros
            pd[i, pl.ds(j * 16, 16)] = zeros

    ones = jnp.ones((16,), jnp.float32)
    m127 = jnp.full((16,), 127, jnp.int32)

    @pl.loop(0, HROWS)
    def _hist(i):
        for j in range(8):
            s = sidx[i, pl.ds(j * 16, 16)]
            d = didx[i, pl.ds(j * 16, 16)]
            plsc.addupdate_scatter(ps, [s >> 7, s & m127], ones)
            plsc.addupdate_scatter(pd, [d >> 7, d & m127], ones)

    pltpu.sync_copy(ps, out_s.at[wid])
    pltpu.sync_copy(pd, out_d.at[wid])


_degree = pl.kernel(
    _degree_body,
    out_type=(
        jax.ShapeDtypeStruct((NW, HROWS, 128), jnp.float32),
        jax.ShapeDtypeStruct((NW, HROWS, 128), jnp.float32),
    ),
    mesh=_MESH,
    compiler_params=_SC_PARAMS,
    scratch_types=[
        pltpu.VMEM((HROWS, 128), jnp.int32),
        pltpu.VMEM((HROWS, 128), jnp.int32),
        pltpu.VMEM((HROWS, 128), jnp.float32),
        pltpu.VMEM((HROWS, 128), jnp.float32),
    ],
)


def _agg_body(table, src_hbm, dst_hbm, out, shared, si0, si1, di0, di1,
              rows0, rows1, semg0, semg1, semis1, semid1, semis0, semid0,
              semsc0, semsc1):
    cid = lax.axis_index("c")
    sid = lax.axis_index("s")
    wid = cid * NS + sid

    zeros = jnp.zeros((16,), jnp.float32)

    @pl.loop(0, CHUNK)
    def _z(i):
        for j in range(8):
            rows0[i, pl.ds(j * 16, 16)] = zeros

    @pl.loop(0, 8)
    def _zs(m):
        ch = sid + m * NS

        @pl.when(ch * 80 < ACC_ROWS)
        def _():
            pltpu.sync_copy(rows0.at[pl.ds(0, 80)], shared.at[pl.ds(ch * 80, 80)])

    plsc.subcore_barrier()

    bufs = (rows0, rows1)
    semg = (semg0, semg1)
    semsc = (semsc0, semsc1)

    def startG(b, sibuf, row):
        pltpu.async_copy(table.at[sibuf.at[row]], bufs[b], semg[b])

    def waitG(b, sibuf, row):
        pltpu.make_async_copy(table.at[sibuf.at[row]], bufs[b], semg[b]).wait()

    def startS(b, dbuf, row):
        pltpu.async_copy(bufs[b], shared.at[dbuf.at[row]], semsc[b], add=True)

    def waitS(b, dbuf, row):
        pltpu.make_async_copy(bufs[b], shared.at[dbuf.at[row]], semsc[b]).wait()

    pltpu.sync_copy(src_hbm.at[wid, 0], si0)
    pltpu.async_copy(dst_hbm.at[wid, 0], di0, semid0)
    startG(0, si0, 0)

    @pl.loop(0, 5)
    def _pair(it):
        for r2 in range(16):
            c_next = it * 16 + r2 + 1
            b = r2 % 2
            ob = 1 - b
            dcur = di0 if r2 < 8 else di1
            scur = si0 if r2 < 8 else si1
            ridx = r2 % 8
            if r2 == 0:
                pltpu.make_async_copy(dst_hbm.at[wid, 0], di0, semid0).wait()
            if r2 == 1:
                pltpu.async_copy(src_hbm.at[wid, 2 * it + 1], si1, semis1)
                pltpu.async_copy(dst_hbm.at[wid, 2 * it + 1], di1, semid1)
            if r2 == 6:
                pltpu.make_async_copy(src_hbm.at[wid, 0], si1, semis1).wait()
            if r2 == 8:
                pltpu.make_async_copy(dst_hbm.at[wid, 0], di1, semid1).wait()
            if r2 == 9:
                @pl.when(2 * it + 2 < NGRP)
                def _():
                    pltpu.async_copy(src_hbm.at[wid, 2 * it + 2], si0, semis0)
                    pltpu.async_copy(dst_hbm.at[wid, 2 * it + 2], di0, semid0)

            if r2 == 0:
                @pl.when(it > 0)
                def _():
                    waitS(ob, di1, 7)
            else:
                pdcur = di0 if (r2 - 1) < 8 else di1
                waitS(ob, pdcur, (r2 - 1) % 8)

            if r2 < 7:
                startG(ob, scur, ridx + 1)
            elif r2 == 7:
                startG(ob, si1, 0)
            elif r2 < 15:
                startG(ob, si1, ridx + 1)
            else:
                @pl.when(c_next < NGRP * 8)
                def _():
                    pltpu.make_async_copy(src_hbm.at[wid, 0], si0, semis0).wait()
                    startG(ob, si0, 0)

            waitG(b, scur, ridx)
            startS(b, dcur, ridx)

    waitS(1, di1, 7)
    plsc.subcore_barrier()

    @pl.loop(0, 8)
    def _fl(m):
        ch = sid + m * NS

        @pl.when(ch < NFL)
        def _():
            pltpu.sync_copy(shared.at[pl.ds(ch * 80, 80)], rows0.at[pl.ds(0, 80)])
            pltpu.sync_copy(rows0.at[pl.ds(0, 80)], out.at[cid, pl.ds(ch * 80, 80)])


_aggregate = pl.kernel(
    _agg_body,
    out_type=jax.ShapeDtypeStruct((NC, N, D), jnp.float32),
    mesh=_MESH,
    compiler_params=_SC_PARAMS,
    scratch_types=[
        pltpu.VMEM_SHARED((ACC_ROWS, D), jnp.float32),
        pltpu.VMEM((8, CHUNK), jnp.int32),
        pltpu.VMEM((8, CHUNK), jnp.int32),
        pltpu.VMEM((8, CHUNK), jnp.int32),
        pltpu.VMEM((8, CHUNK), jnp.int32),
        pltpu.VMEM((CHUNK, D), jnp.float32),
        pltpu.VMEM((CHUNK, D), jnp.float32),
        pltpu.SemaphoreType.DMA,
        pltpu.SemaphoreType.DMA,
        pltpu.SemaphoreType.DMA,
        pltpu.SemaphoreType.DMA,
        pltpu.SemaphoreType.DMA,
        pltpu.SemaphoreType.DMA,
        pltpu.SemaphoreType.DMA,
        pltpu.SemaphoreType.DMA,
    ],
)


_RB = 1000
_NB = N // _RB
_SRC_PAD_COUNT = float(NW * PAD)


def _scale_body(x_ref, ds_ref, dd_ref, xs_ref, ns_ref, nd_ref):
    ds = jnp.sum(ds_ref[...], axis=0)[:N].reshape(N, 1)
    dd = jnp.sum(dd_ref[...], axis=0)[:N].reshape(N, 1)
    row0 = lax.broadcasted_iota(jnp.int32, (N, 1), 0) == 0
    ds = ds - jnp.where(row0, _SRC_PAD_COUNT, 0.0)
    ns = lax.rsqrt(jnp.maximum(ds, 1.0))
    nd = lax.rsqrt(jnp.maximum(dd, 1.0))
    xs_ref[...] = x_ref[...] * ns
    ns_ref[...] = ns
    nd_ref[...] = nd


def _scale(x, ds_t, dd_t):
    return pl.pallas_call(
        _scale_body,
        out_shape=[
            jax.ShapeDtypeStruct((N, D), jnp.float32),
            jax.ShapeDtypeStruct((N, 1), jnp.float32),
            jax.ShapeDtypeStruct((N, 1), jnp.float32),
        ],
    )(x, ds_t, dd_t)


def _dense1_body(p1a, p1b, ns, nd, w1, b1, w2, h1_o, t2_o):
    agg = (p1a[0] + p1b[0]) * nd[...]
    h1 = jnp.maximum(
        jnp.dot(agg, w1[...], preferred_element_type=jnp.float32) + b1[...], 0.0
    )
    h1_o[...] = h1
    t2_o[...] = ns[...] * jnp.dot(h1, w2[...], preferred_element_type=jnp.float32)


def _dense1(p1a, p1b, ns, nd, w1, b1, w2):
    return pl.pallas_call(
        _dense1_body,
        grid=(_NB,),
        in_specs=[
            pl.BlockSpec((1, _RB, D), lambda i: (0, i, 0)),
            pl.BlockSpec((1, _RB, D), lambda i: (1, i, 0)),
            pl.BlockSpec((_RB, 1), lambda i: (i, 0)),
            pl.BlockSpec((_RB, 1), lambda i: (i, 0)),
            pl.BlockSpec((D, D), lambda i: (0, 0)),
            pl.BlockSpec((1, D), lambda i: (0, 0)),
            pl.BlockSpec((D, D), lambda i: (0, 0)),
        ],
        out_specs=[
            pl.BlockSpec((_RB, D), lambda i: (i, 0)),
            pl.BlockSpec((_RB, D), lambda i: (i, 0)),
        ],
        out_shape=[
            jax.ShapeDtypeStruct((N, D), jnp.float32),
            jax.ShapeDtypeStruct((N, D), jnp.float32),
        ],
    )(p1a, p1b, ns, nd, w1, b1, w2)


def _dense2_body(p2a, p2b, nd, b2, h2_o):
    h2_o[...] = (p2a[0] + p2b[0]) * nd[...] + b2[...]


def _dense2(p2a, p2b, nd, b2):
    return pl.pallas_call(
        _dense2_body,
        grid=(_NB,),
        in_specs=[
            pl.BlockSpec((1, _RB, D), lambda i: (0, i, 0)),
            pl.BlockSpec((1, _RB, D), lambda i: (1, i, 0)),
            pl.BlockSpec((_RB, 1), lambda i: (i, 0)),
            pl.BlockSpec((1, D), lambda i: (0, 0)),
        ],
        out_specs=pl.BlockSpec((_RB, D), lambda i: (i, 0)),
        out_shape=jax.ShapeDtypeStruct((N, D), jnp.float32),
    )(p2a, p2b, nd, b2)


@jax.jit
def kernel(feats, edge_index, W1, b1, W2, b2):
    src = edge_index[0].reshape(NW, EPW)
    dst = edge_index[1].reshape(NW, EPW)
    src_p = jnp.pad(src, ((0, 0), (0, PAD)), constant_values=0)
    dst_p = jnp.pad(dst, ((0, 0), (0, PAD)), constant_values=TRASH)
    src4 = src_p.reshape(NW, NGRP, 8, CHUNK)
    dst4 = dst_p.reshape(NW, NGRP, 8, CHUNK)
    src_h = src_p.reshape(NW, HROWS, 128)
    dst_h = dst_p.reshape(NW, HROWS, 128)

    dS_p, dD_p = _degree(src_h, dst_h)
    dS_t = dS_p.reshape(NW, HROWS * 128)
    dD_t = dD_p.reshape(NW, HROWS * 128)

    xs, ns, nd = _scale(feats, dS_t, dD_t)

    p1 = _aggregate(xs, src4, dst4)
    h1, t2 = _dense1(p1, p1, ns, nd, W1, b1.reshape(1, D), W2)

    p2 = _aggregate(t2, src4, dst4)
    h2 = _dense2(p2, p2, nd, b2.reshape(1, D))
    return (h1, h2)


# speedup vs baseline: 1.8122x; 1.0060x over previous
import jax
import jax.numpy as jnp
from jax import lax
from jax.experimental import pallas as pl
from jax.experimental.pallas import tpu as pltpu
from jax.experimental.pallas import tpu_sc as plsc

N = 10000
E = 320000
D = 128

NC = 2
NS = 16
NW = NC * NS
EPW = E // NW
CHUNK = 128
NGRP = 10
EPW_P = NGRP * 8 * CHUNK
PAD = EPW_P - EPW
ACC_ROWS = 10048
TRASH = 10040
HROWS = 80
NFL = 125

_MESH = plsc.VectorSubcoreMesh(
    core_axis_name="c", subcore_axis_name="s", num_cores=NC, num_subcores=NS
)
_SC_PARAMS = pltpu.CompilerParams(needs_layout_passes=False)


def _degree_body(src_hbm, dst_hbm, out_s, out_d, sidx, didx, ps, pd):
    cid = lax.axis_index("c")
    sid = lax.axis_index("s")
    wid = cid * NS + sid
    pltpu.sync_copy(src_hbm.at[wid], sidx)
    pltpu.sync_copy(dst_hbm.at[wid], didx)

    zeros = jnp.zeros((16,), jnp.float32)

    @pl.loop(0, HROWS)
    def _zero(i):
        for j in range(8):
            ps[i, pl.ds(j * 16, 16)] = zeros
            pd[i, pl.ds(j * 16, 16)] = zeros

    ones = jnp.ones((16,), jnp.float32)
    m127 = jnp.full((16,), 127, jnp.int32)

    @pl.loop(0, HROWS)
    def _hist(i):
        for j in range(8):
            s = sidx[i, pl.ds(j * 16, 16)]
            d = didx[i, pl.ds(j * 16, 16)]
            plsc.addupdate_scatter(ps, [s >> 7, s & m127], ones)
            plsc.addupdate_scatter(pd, [d >> 7, d & m127], ones)

    pltpu.sync_copy(ps, out_s.at[wid])
    pltpu.sync_copy(pd, out_d.at[wid])


_degree = pl.kernel(
    _degree_body,
    out_type=(
        jax.ShapeDtypeStruct((NW, HROWS, 128), jnp.float32),
        jax.ShapeDtypeStruct((NW, HROWS, 128), jnp.float32),
    ),
    mesh=_MESH,
    compiler_params=_SC_PARAMS,
    scratch_types=[
        pltpu.VMEM((HROWS, 128), jnp.int32),
        pltpu.VMEM((HROWS, 128), jnp.int32),
        pltpu.VMEM((HROWS, 128), jnp.float32),
        pltpu.VMEM((HROWS, 128), jnp.float32),
    ],
)


def _agg_body(table, src_hbm, dst_hbm, out, shared, si0, si1, di0, di1,
              rows0, rows1, semg0, semg1, semis1, semid1, semis0, semid0,
              semsc0, semsc1):
    cid = lax.axis_index("c")
    sid = lax.axis_index("s")
    wid = cid * NS + sid

    zeros = jnp.zeros((16,), jnp.float32)

    @pl.loop(0, CHUNK)
    def _z(i):
        for j in range(8):
            rows0[i, pl.ds(j * 16, 16)] = zeros

    for m in range(8):
        ch = sid + m * NS

        @pl.when(ch * 80 < ACC_ROWS)
        def _():
            pltpu.async_copy(
                rows0.at[pl.ds(0, 80)], shared.at[pl.ds(ch * 80, 80)], semis0
            )

    for m in range(8):
        ch = sid + m * NS

        @pl.when(ch * 80 < ACC_ROWS)
        def _():
            pltpu.make_async_copy(
                rows0.at[pl.ds(0, 80)], shared.at[pl.ds(ch * 80, 80)], semis0
            ).wait()

    plsc.subcore_barrier()

    bufs = (rows0, rows1)
    semg = (semg0, semg1)
    semsc = (semsc0, semsc1)

    def startG(b, sibuf, row):
        pltpu.async_copy(table.at[sibuf.at[row]], bufs[b], semg[b])

    def waitG(b, sibuf, row):
        pltpu.make_async_copy(table.at[sibuf.at[row]], bufs[b], semg[b]).wait()

    def startS(b, dbuf, row):
        pltpu.async_copy(bufs[b], shared.at[dbuf.at[row]], semsc[b], add=True)

    def waitS(b, dbuf, row):
        pltpu.make_async_copy(bufs[b], shared.at[dbuf.at[row]], semsc[b]).wait()

    pltpu.sync_copy(src_hbm.at[wid, 0], si0)
    pltpu.async_copy(dst_hbm.at[wid, 0], di0, semid0)
    startG(0, si0, 0)

    @pl.loop(0, 5)
    def _pair(it):
        for r2 in range(16):
            c_next = it * 16 + r2 + 1
            b = r2 % 2
            ob = 1 - b
            dcur = di0 if r2 < 8 else di1
            scur = si0 if r2 < 8 else si1
            ridx = r2 % 8
            if r2 == 0:
                pltpu.make_async_copy(dst_hbm.at[wid, 0], di0, semid0).wait()
            if r2 == 1:
                pltpu.async_copy(src_hbm.at[wid, 2 * it + 1], si1, semis1)
                pltpu.async_copy(dst_hbm.at[wid, 2 * it + 1], di1, semid1)
            if r2 == 6:
                pltpu.make_async_copy(src_hbm.at[wid, 0], si1, semis1).wait()
            if r2 == 8:
                pltpu.make_async_copy(dst_hbm.at[wid, 0], di1, semid1).wait()
            if r2 == 9:
                @pl.when(2 * it + 2 < NGRP)
                def _():
                    pltpu.async_copy(src_hbm.at[wid, 2 * it + 2], si0, semis0)
                    pltpu.async_copy(dst_hbm.at[wid, 2 * it + 2], di0, semid0)

            if r2 == 0:
                @pl.when(it > 0)
                def _():
                    waitS(ob, di1, 7)
            else:
                pdcur = di0 if (r2 - 1) < 8 else di1
                waitS(ob, pdcur, (r2 - 1) % 8)

            if r2 < 7:
                startG(ob, scur, ridx + 1)
            elif r2 == 7:
                startG(ob, si1, 0)
            elif r2 < 15:
                startG(ob, si1, ridx + 1)
            else:
                @pl.when(c_next < NGRP * 8)
                def _():
                    pltpu.make_async_copy(src_hbm.at[wid, 0], si0, semis0).wait()
                    startG(ob, si0, 0)

            waitG(b, scur, ridx)
            startS(b, dcur, ridx)

    waitS(1, di1, 7)
    plsc.subcore_barrier()

    fbuf = (rows0, rows1)
    fsem = (semis0, semid0)

    def spill_start(m, ch):
        pltpu.async_copy(
            shared.at[pl.ds(ch * 80, 80)], fbuf[m % 2].at[pl.ds(0, 80)], semg[m % 2]
        )

    def spill_wait(m, ch):
        pltpu.make_async_copy(
            shared.at[pl.ds(ch * 80, 80)], fbuf[m % 2].at[pl.ds(0, 80)], semg[m % 2]
        ).wait()

    def wr_start(m, ch):
        pltpu.async_copy(
            fbuf[m % 2].at[pl.ds(0, 80)], out.at[cid, pl.ds(ch * 80, 80)], fsem[m % 2]
        )

    def wr_wait(m, ch):
        pltpu.make_async_copy(
            fbuf[m % 2].at[pl.ds(0, 80)], out.at[cid, pl.ds(ch * 80, 80)], fsem[m % 2]
        ).wait()

    for m in range(8):
        ch = sid + m * NS

        @pl.when(ch < NFL)
        def _():
            if m >= 2:
                wr_wait(m, sid + (m - 2) * NS)
            spill_start(m, ch)

        @pl.when(ch < NFL)
        def _():
            spill_wait(m, ch)
            wr_start(m, ch)

    for m in (6, 7):
        ch = sid + m * NS

        @pl.when(ch < NFL)
        def _():
            wr_wait(m, ch)


_aggregate = pl.kernel(
    _agg_body,
    out_type=jax.ShapeDtypeStruct((NC, N, D), jnp.float32),
    mesh=_MESH,
    compiler_params=_SC_PARAMS,
    scratch_types=[
        pltpu.VMEM_SHARED((ACC_ROWS, D), jnp.float32),
        pltpu.VMEM((8, CHUNK), jnp.int32),
        pltpu.VMEM((8, CHUNK), jnp.int32),
        pltpu.VMEM((8, CHUNK), jnp.int32),
        pltpu.VMEM((8, CHUNK), jnp.int32),
        pltpu.VMEM((CHUNK, D), jnp.float32),
        pltpu.VMEM((CHUNK, D), jnp.float32),
        pltpu.SemaphoreType.DMA,
        pltpu.SemaphoreType.DMA,
        pltpu.SemaphoreType.DMA,
        pltpu.SemaphoreType.DMA,
        pltpu.SemaphoreType.DMA,
        pltpu.SemaphoreType.DMA,
        pltpu.SemaphoreType.DMA,
        pltpu.SemaphoreType.DMA,
    ],
)


_RB = 1000
_NB = N // _RB
_SRC_PAD_COUNT = float(NW * PAD)


def _scale_body(x_ref, ds_ref, dd_ref, xs_ref, ns_ref, nd_ref):
    ds = jnp.sum(ds_ref[...], axis=0)[:N].reshape(N, 1)
    dd = jnp.sum(dd_ref[...], axis=0)[:N].reshape(N, 1)
    row0 = lax.broadcasted_iota(jnp.int32, (N, 1), 0) == 0
    ds = ds - jnp.where(row0, _SRC_PAD_COUNT, 0.0)
    ns = lax.rsqrt(jnp.maximum(ds, 1.0))
    nd = lax.rsqrt(jnp.maximum(dd, 1.0))
    xs_ref[...] = x_ref[...] * ns
    ns_ref[...] = ns
    nd_ref[...] = nd


def _scale(x, ds_t, dd_t):
    return pl.pallas_call(
        _scale_body,
        out_shape=[
            jax.ShapeDtypeStruct((N, D), jnp.float32),
            jax.ShapeDtypeStruct((N, 1), jnp.float32),
            jax.ShapeDtypeStruct((N, 1), jnp.float32),
        ],
    )(x, ds_t, dd_t)


def _dense1_body(p1a, p1b, ns, nd, w1, b1, w2, h1_o, t2_o):
    agg = (p1a[0] + p1b[0]) * nd[...]
    h1 = jnp.maximum(
        jnp.dot(agg, w1[...], preferred_element_type=jnp.float32) + b1[...], 0.0
    )
    h1_o[...] = h1
    t2_o[...] = ns[...] * jnp.dot(h1, w2[...], preferred_element_type=jnp.float32)


def _dense1(p1a, p1b, ns, nd, w1, b1, w2):
    return pl.pallas_call(
        _dense1_body,
        grid=(_NB,),
        in_specs=[
            pl.BlockSpec((1, _RB, D), lambda i: (0, i, 0)),
            pl.BlockSpec((1, _RB, D), lambda i: (1, i, 0)),
            pl.BlockSpec((_RB, 1), lambda i: (i, 0)),
            pl.BlockSpec((_RB, 1), lambda i: (i, 0)),
            pl.BlockSpec((D, D), lambda i: (0, 0)),
            pl.BlockSpec((1, D), lambda i: (0, 0)),
            pl.BlockSpec((D, D), lambda i: (0, 0)),
        ],
        out_specs=[
            pl.BlockSpec((_RB, D), lambda i: (i, 0)),
            pl.BlockSpec((_RB, D), lambda i: (i, 0)),
        ],
        out_shape=[
            jax.ShapeDtypeStruct((N, D), jnp.float32),
            jax.ShapeDtypeStruct((N, D), jnp.float32),
        ],
    )(p1a, p1b, ns, nd, w1, b1, w2)


def _dense2_body(p2a, p2b, nd, b2, h2_o):
    h2_o[...] = (p2a[0] + p2b[0]) * nd[...] + b2[...]


def _dense2(p2a, p2b, nd, b2):
    return pl.pallas_call(
        _dense2_body,
        grid=(_NB,),
        in_specs=[
            pl.BlockSpec((1, _RB, D), lambda i: (0, i, 0)),
            pl.BlockSpec((1, _RB, D), lambda i: (1, i, 0)),
            pl.BlockSpec((_RB, 1), lambda i: (i, 0)),
            pl.BlockSpec((1, D), lambda i: (0, 0)),
        ],
        out_specs=pl.BlockSpec((_RB, D), lambda i: (i, 0)),
        out_shape=jax.ShapeDtypeStruct((N, D), jnp.float32),
    )(p2a, p2b, nd, b2)


@jax.jit
def kernel(feats, edge_index, W1, b1, W2, b2):
    src = edge_index[0].reshape(NW, EPW)
    dst = edge_index[1].reshape(NW, EPW)
    src_p = jnp.pad(src, ((0, 0), (0, PAD)), constant_values=0)
    dst_p = jnp.pad(dst, ((0, 0), (0, PAD)), constant_values=TRASH)
    src4 = src_p.reshape(NW, NGRP, 8, CHUNK)
    dst4 = dst_p.reshape(NW, NGRP, 8, CHUNK)
    src_h = src_p.reshape(NW, HROWS, 128)
    dst_h = dst_p.reshape(NW, HROWS, 128)

    dS_p, dD_p = _degree(src_h, dst_h)
    dS_t = dS_p.reshape(NW, HROWS * 128)
    dD_t = dD_p.reshape(NW, HROWS * 128)

    xs, ns, nd = _scale(feats, dS_t, dD_t)

    p1 = _aggregate(xs, src4, dst4)
    h1, t2 = _dense1(p1, p1, ns, nd, W1, b1.reshape(1, D), W2)

    p2 = _aggregate(t2, src4, dst4)
    h2 = _dense2(p2, p2, nd, b2.reshape(1, D))
    return (h1, h2)
